# baseline (speedup 1.0000x reference)
import dataclasses

import jax
import jax.numpy as jnp
from jax import lax
from jax.experimental import pallas as pl
from jax.experimental.pallas import tpu as pltpu
from jax.experimental.pallas import tpu_sc as plsc

NB_TF = 4
B, C, H, W = 256, 3, 224, 224
L = 16
NC, NS = 2, 16
NW = NC * NS
KCH = B // L
HPT = H // NW
HALF = H // 2


def _minmax_body(xt_hbm, part_hbm, buf0, buf1, acc_v, stage_v, shared_v,
                 isem0, isem1):
    cid = lax.axis_index("c")
    sid = lax.axis_index("s")
    wid = cid * NS + sid

    bufs = (buf0, buf1)
    isems = (isem0, isem1)

    def load(j, b):
        c, r = divmod(j, HPT)
        pltpu.async_copy(xt_hbm.at[c, wid * HPT + r], bufs[b], isems[b])

    def wait_load(b):
        pltpu.make_async_copy(xt_hbm.at[0, 0], bufs[b], isems[b]).wait()

    for c in range(C):
        @pl.loop(0, KCH)
        def _(k):
            acc_v[c, 0, pl.ds(k * L, L)] = jnp.full((L,), jnp.inf, jnp.float32)
            acc_v[c, 1, pl.ds(k * L, L)] = jnp.full((L,), -jnp.inf, jnp.float32)

    load(0, 0)
    for j in range(C * HPT):
        b = j % 2
        c = j // HPT
        wait_load(b)
        if j + 1 < C * HPT:
            load(j + 1, 1 - b)

        @pl.loop(0, KCH)
        def _(k):
            sl = pl.ds(k * L, L)

            @pl.loop(0, H, step=16)
            def _(w0):
                vs = [bufs[b][w0 + i, sl] for i in range(16)]
                mn = vs[0]
                mx = vs[0]
                for v in vs[1:]:
                    mn = jnp.minimum(mn, v)
                    mx = jnp.maximum(mx, v)
                acc_v[c, 0, sl] = jnp.minimum(acc_v[c, 0, sl], mn)
                acc_v[c, 1, sl] = jnp.maximum(acc_v[c, 1, sl], mx)

    pltpu.sync_copy(acc_v, shared_v.at[sid])
    plsc.subcore_barrier()

    @pl.when(sid == 0)
    def _():
        @pl.loop(0, NS)
        def _(i):
            pltpu.sync_copy(shared_v.at[i], stage_v)
            for c in range(C):
                @pl.loop(0, KCH)
                def _(k):
                    sl = pl.ds(k * L, L)
                    acc_v[c, 0, sl] = jnp.minimum(acc_v[c, 0, sl], stage_v[c, 0, sl])
                    acc_v[c, 1, sl] = jnp.maximum(acc_v[c, 1, sl], stage_v[c, 1, sl])

        pltpu.sync_copy(acc_v, part_hbm.at[cid])


BAND = HALF // 2
ITEMS_PER_C = 2 * HALF // NW


def _apply_body(xt_hbm, tf_hbm, part_hbm, o_hbm,
                bufA0, bufB0, bufA1, bufB1, cons_v, stage2_v, tf_v,
                iA0, iB0, iA1, iB1, oA0, oB0, oA1, oB1):
    cid = lax.axis_index("c")
    sid = lax.axis_index("s")
    wid = cid * NS + sid

    bufsA = (bufA0, bufA1)
    bufsB = (bufB0, bufB1)
    isemsA = (iA0, iA1)
    isemsB = (iB0, iB1)
    osemsA = (oA0, oA1)
    osemsB = (oB0, oB1)

    pltpu.sync_copy(tf_hbm, tf_v)
    pltpu.sync_copy(part_hbm, stage2_v)
    for c in range(C):
        @pl.loop(0, KCH)
        def _(k):
            sl = pl.ds(k * L, L)
            mn = jnp.minimum(stage2_v[0, c, 0, sl], stage2_v[1, c, 0, sl])
            mx = jnp.maximum(stage2_v[0, c, 1, sl], stage2_v[1, c, 1, sl])
            sc = jnp.full((L,), 1.0, jnp.float32) / jnp.maximum(
                mx - mn, jnp.full((L,), 1e-6, jnp.float32))
            cons_v[c, 0, sl] = mn
            cons_v[c, 1, sl] = sc

    NITEMS = C * ITEMS_PER_C

    def item_coords(m):
        c = m // ITEMS_PER_C
        q = wid + NW * (m % ITEMS_PER_C)
        hp = lax.shift_right_logical(q, 1)
        g = lax.bitwise_and(q, 1)
        return c, hp, g

    def _move(m, b, out):
        c, hp, g = item_coords(m)
        lo = pl.ds(g * BAND, BAND)
        hi = pl.ds(W - BAND - g * BAND, BAND)
        for row_hbm, buf, sems in (
            (hp, bufsA[b], (osemsA if out else isemsA)[b]),
            (H - 1 - hp, bufsB[b], (osemsB if out else isemsB)[b]),
        ):
            hbm = o_hbm if out else xt_hbm
            if out:
                pltpu.async_copy(buf.at[pl.ds(0, BAND)], hbm.at[c, row_hbm, lo], sems)
                pltpu.async_copy(buf.at[pl.ds(BAND, BAND)], hbm.at[c, row_hbm, hi], sems)
            else:
                pltpu.async_copy(hbm.at[c, row_hbm, lo], buf.at[pl.ds(0, BAND)], sems)
                pltpu.async_copy(hbm.at[c, row_hbm, hi], buf.at[pl.ds(BAND, BAND)], sems)

    def load(m, b):
        _move(m, b, out=False)

    def store(m, b):
        _move(m, b, out=True)

    def wait_load(b):
        pltpu.make_async_copy(xt_hbm.at[0, 0, pl.ds(0, 2 * BAND)], bufsA[b], isemsA[b]).wait()
        pltpu.make_async_copy(xt_hbm.at[0, 0, pl.ds(0, 2 * BAND)], bufsB[b], isemsB[b]).wait()

    def wait_store(b):
        pltpu.make_async_copy(bufsA[b], o_hbm.at[0, 0, pl.ds(0, 2 * BAND)], osemsA[b]).wait()
        pltpu.make_async_copy(bufsB[b], o_hbm.at[0, 0, pl.ds(0, 2 * BAND)], osemsB[b]).wait()

    load(0, 0)
    for m in range(NITEMS):
        b = m % 2
        c = m // ITEMS_PER_C
        wait_load(b)

        @pl.loop(0, KCH)
        def _(kk):
            sl = pl.ds(kk * L, L)
            tv = tf_v[sl]
            m1 = tv == 1
            m2 = tv == 2
            m3 = tv == 3
            mn = cons_v[c, 0, sl]
            sc = cons_v[c, 1, sl]
            bA = bufsA[b]
            bB = bufsB[b]

            @pl.loop(0, BAND, step=4)
            def _(w0):
                for i in range(4):
                    w = w0 + i
                    mw = 2 * BAND - 1 - w
                    aw = bA[w, sl]
                    am = bA[mw, sl]
                    bw = bB[w, sl]
                    bm = bB[mw, sl]
                    bA[w, sl] = jnp.where(
                        m3, (aw - mn) * sc,
                        jnp.where(m2, bw, jnp.where(m1, am, aw)))
                    bA[mw, sl] = jnp.where(
                        m3, (am - mn) * sc,
                        jnp.where(m2, bm, jnp.where(m1, aw, am)))
                    bB[w, sl] = jnp.where(
                        m3, (bw - mn) * sc,
                        jnp.where(m2, aw, jnp.where(m1, bm, bw)))
                    bB[mw, sl] = jnp.where(
                        m3, (bm - mn) * sc,
                        jnp.where(m2, am, jnp.where(m1, bw, bm)))

        if m >= 1:
            wait_store(1 - b)
        if m + 1 < NITEMS:
            load(m + 1, 1 - b)
        store(m, b)

    wait_store((NITEMS - 1) % 2)


@jax.jit
def kernel(x, sampled_tf):
    xt = jnp.transpose(x, (1, 2, 3, 0))
    mesh = plsc.VectorSubcoreMesh(
        core_axis_name="c", subcore_axis_name="s", num_cores=NC, num_subcores=NS
    )
    cp = pltpu.CompilerParams()
    if "needs_layout_passes" in pltpu.CompilerParams.__dataclass_fields__:
        cp = dataclasses.replace(cp, needs_layout_passes=False)
    part = pl.kernel(
        _minmax_body,
        out_type=jax.ShapeDtypeStruct((NC, C, 2, B), jnp.float32),
        mesh=mesh,
        scratch_types=[
            pltpu.VMEM((W, B), jnp.float32),
            pltpu.VMEM((W, B), jnp.float32),
            pltpu.VMEM((C, 2, B), jnp.float32),
            pltpu.VMEM((C, 2, B), jnp.float32),
            pltpu.VMEM_SHARED((NS, C, 2, B), jnp.float32),
            pltpu.SemaphoreType.DMA,
            pltpu.SemaphoreType.DMA,
        ],
        compiler_params=cp,
    )(xt)
    ot = pl.kernel(
        _apply_body,
        out_type=jax.ShapeDtypeStruct((C, H, W, B), jnp.float32),
        mesh=mesh,
        scratch_types=[
            pltpu.VMEM((2 * BAND, B), jnp.float32),
            pltpu.VMEM((2 * BAND, B), jnp.float32),
            pltpu.VMEM((2 * BAND, B), jnp.float32),
            pltpu.VMEM((2 * BAND, B), jnp.float32),
            pltpu.VMEM((C, 2, B), jnp.float32),
            pltpu.VMEM((NC, C, 2, B), jnp.float32),
            pltpu.VMEM((B,), jnp.int32),
            pltpu.SemaphoreType.DMA,
            pltpu.SemaphoreType.DMA,
            pltpu.SemaphoreType.DMA,
            pltpu.SemaphoreType.DMA,
            pltpu.SemaphoreType.DMA,
            pltpu.SemaphoreType.DMA,
            pltpu.SemaphoreType.DMA,
            pltpu.SemaphoreType.DMA,
        ],
        compiler_params=cp,
    )(xt, sampled_tf, part)
    return jnp.transpose(ot, (3, 0, 1, 2))


# speedup vs baseline: 1.0239x; 1.0239x over previous
import dataclasses

import jax
import jax.numpy as jnp
from jax import lax
from jax.experimental import pallas as pl
from jax.experimental.pallas import tpu as pltpu
from jax.experimental.pallas import tpu_sc as plsc

NB_TF = 4
B, C, H, W = 256, 3, 224, 224
L = 16
NC, NS = 2, 16
NW = NC * NS
KCH = B // L
HPT = H // NW
HALF = H // 2


def _minmax_body(xt_hbm, part_hbm, buf0, buf1, acc_v, stage_v, shared_v,
                 isem0, isem1):
    cid = lax.axis_index("c")
    sid = lax.axis_index("s")
    wid = cid * NS + sid

    bufs = (buf0, buf1)
    isems = (isem0, isem1)

    def load(j, b):
        c, r = divmod(j, HPT)
        pltpu.async_copy(xt_hbm.at[c, wid * HPT + r], bufs[b], isems[b])

    def wait_load(b):
        pltpu.make_async_copy(xt_hbm.at[0, 0], bufs[b], isems[b]).wait()

    for c in range(C):
        @pl.loop(0, KCH)
        def _(k):
            acc_v[c, 0, pl.ds(k * L, L)] = jnp.full((L,), jnp.inf, jnp.float32)
            acc_v[c, 1, pl.ds(k * L, L)] = jnp.full((L,), -jnp.inf, jnp.float32)

    load(0, 0)
    for j in range(C * HPT):
        b = j % 2
        c = j // HPT
        wait_load(b)
        if j + 1 < C * HPT:
            load(j + 1, 1 - b)

        @pl.loop(0, KCH)
        def _(k):
            sl = pl.ds(k * L, L)

            @pl.loop(0, H, step=16)
            def _(w0):
                vs = [bufs[b][w0 + i, sl] for i in range(16)]
                mn = vs[0]
                mx = vs[0]
                for v in vs[1:]:
                    mn = jnp.minimum(mn, v)
                    mx = jnp.maximum(mx, v)
                acc_v[c, 0, sl] = jnp.minimum(acc_v[c, 0, sl], mn)
                acc_v[c, 1, sl] = jnp.maximum(acc_v[c, 1, sl], mx)

    pltpu.sync_copy(acc_v, shared_v.at[sid])
    plsc.subcore_barrier()

    @pl.when(sid == 0)
    def _():
        @pl.loop(0, NS)
        def _(i):
            pltpu.sync_copy(shared_v.at[i], stage_v)
            for c in range(C):
                @pl.loop(0, KCH)
                def _(k):
                    sl = pl.ds(k * L, L)
                    acc_v[c, 0, sl] = jnp.minimum(acc_v[c, 0, sl], stage_v[c, 0, sl])
                    acc_v[c, 1, sl] = jnp.maximum(acc_v[c, 1, sl], stage_v[c, 1, sl])

        pltpu.sync_copy(acc_v, part_hbm.at[cid])


LH = B // 2
ITEMS_PER_C = 2 * HALF // NW


def _apply_body(xt_hbm, tf_hbm, part_hbm, o_hbm,
                bufA0, bufB0, bufA1, bufB1, cons_v, stage2_v, tf_v,
                iA0, iB0, iA1, iB1, oA0, oB0, oA1, oB1):
    cid = lax.axis_index("c")
    sid = lax.axis_index("s")
    wid = cid * NS + sid

    bufsA = (bufA0, bufA1)
    bufsB = (bufB0, bufB1)
    isemsA = (iA0, iA1)
    isemsB = (iB0, iB1)
    osemsA = (oA0, oA1)
    osemsB = (oB0, oB1)

    pltpu.sync_copy(tf_hbm, tf_v)
    pltpu.sync_copy(part_hbm, stage2_v)
    for c in range(C):
        @pl.loop(0, KCH)
        def _(k):
            sl = pl.ds(k * L, L)
            mn = jnp.minimum(stage2_v[0, c, 0, sl], stage2_v[1, c, 0, sl])
            mx = jnp.maximum(stage2_v[0, c, 1, sl], stage2_v[1, c, 1, sl])
            sc = jnp.full((L,), 1.0, jnp.float32) / jnp.maximum(
                mx - mn, jnp.full((L,), 1e-6, jnp.float32))
            cons_v[c, 0, sl] = mn
            cons_v[c, 1, sl] = sc

    NITEMS = C * ITEMS_PER_C

    def item_coords(m):
        c = m // ITEMS_PER_C
        q = wid + NW * (m % ITEMS_PER_C)
        hp = lax.shift_right_logical(q, 1)
        g = lax.bitwise_and(q, 1)
        return c, hp, g

    def load(m, b):
        c, hp, g = item_coords(m)
        col = pl.ds(g * LH, LH)
        pltpu.async_copy(xt_hbm.at[c, hp, :, col], bufsA[b], isemsA[b])
        pltpu.async_copy(xt_hbm.at[c, H - 1 - hp, :, col], bufsB[b], isemsB[b])

    def store(m, b):
        c, hp, g = item_coords(m)
        col = pl.ds(g * LH, LH)
        pltpu.async_copy(bufsA[b], o_hbm.at[c, hp, :, col], osemsA[b])
        pltpu.async_copy(bufsB[b], o_hbm.at[c, H - 1 - hp, :, col], osemsB[b])

    def wait_load(b):
        pltpu.make_async_copy(xt_hbm.at[0, 0, :, pl.ds(0, LH)], bufsA[b], isemsA[b]).wait()
        pltpu.make_async_copy(xt_hbm.at[0, 0, :, pl.ds(0, LH)], bufsB[b], isemsB[b]).wait()

    def wait_store(b):
        pltpu.make_async_copy(bufsA[b], o_hbm.at[0, 0, :, pl.ds(0, LH)], osemsA[b]).wait()
        pltpu.make_async_copy(bufsB[b], o_hbm.at[0, 0, :, pl.ds(0, LH)], osemsB[b]).wait()

    load(0, 0)
    for m in range(NITEMS):
        b = m % 2
        c, _hp, g = item_coords(m)
        gbase = g * LH
        wait_load(b)

        @pl.loop(0, LH // L)
        def _(kk):
            sl = pl.ds(kk * L, L)
            gsl = pl.ds(gbase + kk * L, L)
            tv = tf_v[gsl]
            m1 = tv == 1
            m2 = tv == 2
            m3 = tv == 3
            mn = cons_v[c, 0, gsl]
            sc = cons_v[c, 1, gsl]
            bA = bufsA[b]
            bB = bufsB[b]

            @pl.loop(0, HALF, step=8)
            def _(w0):
                for i in range(8):
                    w = w0 + i
                    mw = W - 1 - w
                    aw = bA[w, sl]
                    am = bA[mw, sl]
                    bw = bB[w, sl]
                    bm = bB[mw, sl]
                    bA[w, sl] = jnp.where(
                        m3, (aw - mn) * sc,
                        jnp.where(m2, bw, jnp.where(m1, am, aw)))
                    bA[mw, sl] = jnp.where(
                        m3, (am - mn) * sc,
                        jnp.where(m2, bm, jnp.where(m1, aw, am)))
                    bB[w, sl] = jnp.where(
                        m3, (bw - mn) * sc,
                        jnp.where(m2, aw, jnp.where(m1, bm, bw)))
                    bB[mw, sl] = jnp.where(
                        m3, (bm - mn) * sc,
                        jnp.where(m2, am, jnp.where(m1, bw, bm)))

        if m >= 1:
            wait_store(1 - b)
        if m + 1 < NITEMS:
            load(m + 1, 1 - b)
        store(m, b)

    wait_store((NITEMS - 1) % 2)


@jax.jit
def kernel(x, sampled_tf):
    xt = jnp.transpose(x, (1, 2, 3, 0))
    mesh = plsc.VectorSubcoreMesh(
        core_axis_name="c", subcore_axis_name="s", num_cores=NC, num_subcores=NS
    )
    cp = pltpu.CompilerParams()
    if "needs_layout_passes" in pltpu.CompilerParams.__dataclass_fields__:
        cp = dataclasses.replace(cp, needs_layout_passes=False)
    part = pl.kernel(
        _minmax_body,
        out_type=jax.ShapeDtypeStruct((NC, C, 2, B), jnp.float32),
        mesh=mesh,
        scratch_types=[
            pltpu.VMEM((W, B), jnp.float32),
            pltpu.VMEM((W, B), jnp.float32),
            pltpu.VMEM((C, 2, B), jnp.float32),
            pltpu.VMEM((C, 2, B), jnp.float32),
            pltpu.VMEM_SHARED((NS, C, 2, B), jnp.float32),
            pltpu.SemaphoreType.DMA,
            pltpu.SemaphoreType.DMA,
        ],
        compiler_params=cp,
    )(xt)
    ot = pl.kernel(
        _apply_body,
        out_type=jax.ShapeDtypeStruct((C, H, W, B), jnp.float32),
        mesh=mesh,
        scratch_types=[
            pltpu.VMEM((W, LH), jnp.float32),
            pltpu.VMEM((W, LH), jnp.float32),
            pltpu.VMEM((W, LH), jnp.float32),
            pltpu.VMEM((W, LH), jnp.float32),
            pltpu.VMEM((C, 2, B), jnp.float32),
            pltpu.VMEM((NC, C, 2, B), jnp.float32),
            pltpu.VMEM((B,), jnp.int32),
            pltpu.SemaphoreType.DMA,
            pltpu.SemaphoreType.DMA,
            pltpu.SemaphoreType.DMA,
            pltpu.SemaphoreType.DMA,
            pltpu.SemaphoreType.DMA,
            pltpu.SemaphoreType.DMA,
            pltpu.SemaphoreType.DMA,
            pltpu.SemaphoreType.DMA,
        ],
        compiler_params=cp,
    )(xt, sampled_tf, part)
    return jnp.transpose(ot, (3, 0, 1, 2))


# speedup vs baseline: 1.0307x; 1.0066x over previous
import dataclasses

import jax
import jax.numpy as jnp
from jax import lax
from jax.experimental import pallas as pl
from jax.experimental.pallas import tpu as pltpu
from jax.experimental.pallas import tpu_sc as plsc

NB_TF = 4
B, C, H, W = 256, 3, 224, 224
L = 16
NC, NS = 2, 16
NW = NC * NS
KCH = B // L
HPT = H // NW
HALF = H // 2


def _minmax_body(xt_hbm, part_hbm, buf0, buf1, acc_v, stage_v, shared_v,
                 isem0, isem1):
    cid = lax.axis_index("c")
    sid = lax.axis_index("s")
    wid = cid * NS + sid

    bufs = (buf0, buf1)
    isems = (isem0, isem1)

    def load(j, b):
        c, r = divmod(j, HPT)
        pltpu.async_copy(xt_hbm.at[c, wid * HPT + r], bufs[b], isems[b])

    def wait_load(b):
        pltpu.make_async_copy(xt_hbm.at[0, 0], bufs[b], isems[b]).wait()

    for c in range(C):
        @pl.loop(0, KCH)
        def _(k):
            acc_v[c, 0, pl.ds(k * L, L)] = jnp.full((L,), jnp.inf, jnp.float32)
            acc_v[c, 1, pl.ds(k * L, L)] = jnp.full((L,), -jnp.inf, jnp.float32)

    load(0, 0)
    for j in range(C * HPT):
        b = j % 2
        c = j // HPT
        wait_load(b)
        if j + 1 < C * HPT:
            load(j + 1, 1 - b)

        @pl.loop(0, KCH)
        def _(k):
            sl = pl.ds(k * L, L)

            @pl.loop(0, H, step=16)
            def _(w0):
                vs = [bufs[b][w0 + i, sl] for i in range(16)]
                mn = vs[0]
                mx = vs[0]
                for v in vs[1:]:
                    mn = jnp.minimum(mn, v)
                    mx = jnp.maximum(mx, v)
                acc_v[c, 0, sl] = jnp.minimum(acc_v[c, 0, sl], mn)
                acc_v[c, 1, sl] = jnp.maximum(acc_v[c, 1, sl], mx)

    pltpu.sync_copy(acc_v, shared_v.at[sid])
    plsc.subcore_barrier()

    @pl.when(sid == 0)
    def _():
        @pl.loop(0, NS)
        def _(i):
            pltpu.sync_copy(shared_v.at[i], stage_v)
            for c in range(C):
                @pl.loop(0, KCH)
                def _(k):
                    sl = pl.ds(k * L, L)
                    acc_v[c, 0, sl] = jnp.minimum(acc_v[c, 0, sl], stage_v[c, 0, sl])
                    acc_v[c, 1, sl] = jnp.maximum(acc_v[c, 1, sl], stage_v[c, 1, sl])

        pltpu.sync_copy(acc_v, part_hbm.at[cid])


LH = B // 2
ITEMS_PER_C = 2 * HALF // NW


def _apply_body(xt_hbm, tf_hbm, part_hbm, o_hbm,
                bufA0, bufB0, bufA1, bufB1, cons_v, stage2_v, tf_v,
                iA0, iB0, iA1, iB1, oA0, oB0, oA1, oB1):
    cid = lax.axis_index("c")
    sid = lax.axis_index("s")
    wid = cid * NS + sid

    bufsA = (bufA0, bufA1)
    bufsB = (bufB0, bufB1)
    isemsA = (iA0, iA1)
    isemsB = (iB0, iB1)
    osemsA = (oA0, oA1)
    osemsB = (oB0, oB1)

    pltpu.sync_copy(tf_hbm, tf_v)
    pltpu.sync_copy(part_hbm, stage2_v)
    for c in range(C):
        @pl.loop(0, KCH)
        def _(k):
            sl = pl.ds(k * L, L)
            mn = jnp.minimum(stage2_v[0, c, 0, sl], stage2_v[1, c, 0, sl])
            mx = jnp.maximum(stage2_v[0, c, 1, sl], stage2_v[1, c, 1, sl])
            sc = jnp.full((L,), 1.0, jnp.float32) / jnp.maximum(
                mx - mn, jnp.full((L,), 1e-6, jnp.float32))
            cons_v[c, 0, sl] = mn
            cons_v[c, 1, sl] = sc

    NITEMS = C * ITEMS_PER_C

    def item_coords(m):
        c = m // ITEMS_PER_C
        q = wid + NW * (m % ITEMS_PER_C)
        hp = lax.shift_right_logical(q, 1)
        g = lax.bitwise_and(q, 1)
        return c, hp, g

    def load(m, b):
        c, hp, g = item_coords(m)
        col = pl.ds(g * LH, LH)
        pltpu.async_copy(xt_hbm.at[c, hp, :, col], bufsA[b], isemsA[b])
        pltpu.async_copy(xt_hbm.at[c, H - 1 - hp, :, col], bufsB[b], isemsB[b])

    def store(m, b):
        c, hp, g = item_coords(m)
        col = pl.ds(g * LH, LH)
        pltpu.async_copy(bufsA[b], o_hbm.at[c, hp, :, col], osemsA[b])
        pltpu.async_copy(bufsB[b], o_hbm.at[c, H - 1 - hp, :, col], osemsB[b])

    def wait_load(b):
        pltpu.make_async_copy(xt_hbm.at[0, 0, :, pl.ds(0, LH)], bufsA[b], isemsA[b]).wait()
        pltpu.make_async_copy(xt_hbm.at[0, 0, :, pl.ds(0, LH)], bufsB[b], isemsB[b]).wait()

    def wait_store(b):
        pltpu.make_async_copy(bufsA[b], o_hbm.at[0, 0, :, pl.ds(0, LH)], osemsA[b]).wait()
        pltpu.make_async_copy(bufsB[b], o_hbm.at[0, 0, :, pl.ds(0, LH)], osemsB[b]).wait()

    load(0, 0)
    for m in range(NITEMS):
        b = m % 2
        c, _hp, g = item_coords(m)
        gbase = g * LH
        wait_load(b)

        @pl.loop(0, LH // L)
        def _(kk):
            sl = pl.ds(kk * L, L)
            gsl = pl.ds(gbase + kk * L, L)
            tv = tf_v[gsl]
            m1 = tv == 1
            m2 = tv == 2
            m3 = tv == 3
            mn = cons_v[c, 0, gsl]
            sc = cons_v[c, 1, gsl]
            bA = bufsA[b]
            bB = bufsB[b]

            @pl.loop(0, HALF, step=4)
            def _(w0):
                for i in range(4):
                    w = w0 + i
                    mw = W - 1 - w
                    aw = bA[w, sl]
                    am = bA[mw, sl]
                    bw = bB[w, sl]
                    bm = bB[mw, sl]
                    bA[w, sl] = jnp.where(
                        m3, (aw - mn) * sc,
                        jnp.where(m2, bw, jnp.where(m1, am, aw)))
                    bA[mw, sl] = jnp.where(
                        m3, (am - mn) * sc,
                        jnp.where(m2, bm, jnp.where(m1, aw, am)))
                    bB[w, sl] = jnp.where(
                        m3, (bw - mn) * sc,
                        jnp.where(m2, aw, jnp.where(m1, bm, bw)))
                    bB[mw, sl] = jnp.where(
                        m3, (bm - mn) * sc,
                        jnp.where(m2, am, jnp.where(m1, bw, bm)))

        if m >= 1:
            wait_store(1 - b)
        if m + 1 < NITEMS:
            load(m + 1, 1 - b)
        store(m, b)

    wait_store((NITEMS - 1) % 2)


@jax.jit
def kernel(x, sampled_tf):
    xt = jnp.transpose(x, (1, 2, 3, 0))
    mesh = plsc.VectorSubcoreMesh(
        core_axis_name="c", subcore_axis_name="s", num_cores=NC, num_subcores=NS
    )
    cp = pltpu.CompilerParams()
    if "needs_layout_passes" in pltpu.CompilerParams.__dataclass_fields__:
        cp = dataclasses.replace(cp, needs_layout_passes=False)
    part = pl.kernel(
        _minmax_body,
        out_type=jax.ShapeDtypeStruct((NC, C, 2, B), jnp.float32),
        mesh=mesh,
        scratch_types=[
            pltpu.VMEM((W, B), jnp.float32),
            pltpu.VMEM((W, B), jnp.float32),
            pltpu.VMEM((C, 2, B), jnp.float32),
            pltpu.VMEM((C, 2, B), jnp.float32),
            pltpu.VMEM_SHARED((NS, C, 2, B), jnp.float32),
            pltpu.SemaphoreType.DMA,
            pltpu.SemaphoreType.DMA,
        ],
        compiler_params=cp,
    )(xt)
    ot = pl.kernel(
        _apply_body,
        out_type=jax.ShapeDtypeStruct((C, H, W, B), jnp.float32),
        mesh=mesh,
        scratch_types=[
            pltpu.VMEM((W, LH), jnp.float32),
            pltpu.VMEM((W, LH), jnp.float32),
            pltpu.VMEM((W, LH), jnp.float32),
            pltpu.VMEM((W, LH), jnp.float32),
            pltpu.VMEM((C, 2, B), jnp.float32),
            pltpu.VMEM((NC, C, 2, B), jnp.float32),
            pltpu.VMEM((B,), jnp.int32),
            pltpu.SemaphoreType.DMA,
            pltpu.SemaphoreType.DMA,
            pltpu.SemaphoreType.DMA,
            pltpu.SemaphoreType.DMA,
            pltpu.SemaphoreType.DMA,
            pltpu.SemaphoreType.DMA,
            pltpu.SemaphoreType.DMA,
            pltpu.SemaphoreType.DMA,
        ],
        compiler_params=cp,
    )(xt, sampled_tf, part)
    return jnp.transpose(ot, (3, 0, 1, 2))


# speedup vs baseline: 1.1358x; 1.1020x over previous
import dataclasses

import jax
import jax.numpy as jnp
from jax import lax
from jax.experimental import pallas as pl
from jax.experimental.pallas import tpu as pltpu
from jax.experimental.pallas import tpu_sc as plsc

NB_TF = 4
B, C, H, W = 256, 3, 224, 224
L = 16
NC, NS = 2, 16
NW = NC * NS
KCH = B // L
HPT = H // NW
HALF = H // 2


def _minmax_body(xt_hbm, part_hbm, buf0, buf1, acc_v, stage_v, shared_v,
                 isem0, isem1):
    cid = lax.axis_index("c")
    sid = lax.axis_index("s")
    wid = cid * NS + sid

    bufs = (buf0, buf1)
    isems = (isem0, isem1)

    def load(j, b):
        pltpu.async_copy(xt_hbm.at[0, wid * HPT + j], bufs[b], isems[b])

    def wait_load(b):
        pltpu.make_async_copy(xt_hbm.at[0, 0], bufs[b], isems[b]).wait()

    @pl.loop(0, KCH)
    def _(k):
        acc_v[0, pl.ds(k * L, L)] = jnp.full((L,), jnp.inf, jnp.float32)
        acc_v[1, pl.ds(k * L, L)] = jnp.full((L,), -jnp.inf, jnp.float32)

    load(0, 0)
    for j in range(HPT):
        b = j % 2
        wait_load(b)
        if j + 1 < HPT:
            load(j + 1, 1 - b)

        @pl.loop(0, KCH)
        def _(k):
            sl = pl.ds(k * L, L)

            @pl.loop(0, H, step=16)
            def _(w0):
                vs = [bufs[b][w0 + i, sl] for i in range(16)]
                mn = vs[0]
                mx = vs[0]
                for v in vs[1:]:
                    mn = jnp.minimum(mn, v)
                    mx = jnp.maximum(mx, v)
                acc_v[0, sl] = jnp.minimum(acc_v[0, sl], mn)
                acc_v[1, sl] = jnp.maximum(acc_v[1, sl], mx)

    pltpu.sync_copy(acc_v, shared_v.at[sid])
    plsc.subcore_barrier()

    @pl.when(sid == 0)
    def _():
        @pl.loop(0, NS)
        def _(i):
            pltpu.sync_copy(shared_v.at[i], stage_v)

            @pl.loop(0, KCH)
            def _(k):
                sl = pl.ds(k * L, L)
                acc_v[0, sl] = jnp.minimum(acc_v[0, sl], stage_v[0, sl])
                acc_v[1, sl] = jnp.maximum(acc_v[1, sl], stage_v[1, sl])

        pltpu.sync_copy(acc_v, part_hbm.at[cid])


def _tc_minmax_body(x_ref, o_ref):
    h = pl.program_id(1)
    v = x_ref[0]
    mn = jnp.min(v, axis=(0, 1))
    mx = jnp.max(v, axis=(0, 1))

    @pl.when(h == 0)
    def _():
        o_ref[0, 0, :] = mn
        o_ref[0, 1, :] = mx

    @pl.when(h > 0)
    def _():
        o_ref[0, 0, :] = jnp.minimum(o_ref[0, 0, :], mn)
        o_ref[0, 1, :] = jnp.maximum(o_ref[0, 1, :], mx)


LH = B // 2
ITEMS_PER_C = 2 * HALF // NW


def _apply_body(xt_hbm, tf_hbm, psc_hbm, ptc_hbm, o_hbm,
                bufA0, bufB0, bufA1, bufB1, cons_v, stage_sc, stage_tc, tf_v,
                iA0, iB0, iA1, iB1, oA0, oB0, oA1, oB1):
    cid = lax.axis_index("c")
    sid = lax.axis_index("s")
    wid = cid * NS + sid

    bufsA = (bufA0, bufA1)
    bufsB = (bufB0, bufB1)
    isemsA = (iA0, iA1)
    isemsB = (iB0, iB1)
    osemsA = (oA0, oA1)
    osemsB = (oB0, oB1)

    pltpu.sync_copy(tf_hbm, tf_v)
    pltpu.sync_copy(psc_hbm, stage_sc)
    pltpu.sync_copy(ptc_hbm, stage_tc)

    @pl.loop(0, KCH)
    def _(k):
        sl = pl.ds(k * L, L)
        ones = jnp.full((L,), 1.0, jnp.float32)
        eps = jnp.full((L,), 1e-6, jnp.float32)
        mn = jnp.minimum(stage_sc[0, 0, sl], stage_sc[1, 0, sl])
        mx = jnp.maximum(stage_sc[0, 1, sl], stage_sc[1, 1, sl])
        cons_v[0, 0, sl] = mn
        cons_v[0, 1, sl] = ones / jnp.maximum(mx - mn, eps)
        for c in (1, 2):
            mn = stage_tc[c - 1, 0, sl]
            mx = stage_tc[c - 1, 1, sl]
            cons_v[c, 0, sl] = mn
            cons_v[c, 1, sl] = ones / jnp.maximum(mx - mn, eps)

    NITEMS = C * ITEMS_PER_C

    def item_coords(m):
        c = m // ITEMS_PER_C
        q = wid + NW * (m % ITEMS_PER_C)
        hp = lax.shift_right_logical(q, 1)
        g = lax.bitwise_and(q, 1)
        return c, hp, g

    def load(m, b):
        c, hp, g = item_coords(m)
        col = pl.ds(g * LH, LH)
        pltpu.async_copy(xt_hbm.at[c, hp, :, col], bufsA[b], isemsA[b])
        pltpu.async_copy(xt_hbm.at[c, H - 1 - hp, :, col], bufsB[b], isemsB[b])

    def store(m, b):
        c, hp, g = item_coords(m)
        col = pl.ds(g * LH, LH)
        pltpu.async_copy(bufsA[b], o_hbm.at[c, hp, :, col], osemsA[b])
        pltpu.async_copy(bufsB[b], o_hbm.at[c, H - 1 - hp, :, col], osemsB[b])

    def wait_load(b):
        pltpu.make_async_copy(xt_hbm.at[0, 0, :, pl.ds(0, LH)], bufsA[b], isemsA[b]).wait()
        pltpu.make_async_copy(xt_hbm.at[0, 0, :, pl.ds(0, LH)], bufsB[b], isemsB[b]).wait()

    def wait_store(b):
        pltpu.make_async_copy(bufsA[b], o_hbm.at[0, 0, :, pl.ds(0, LH)], osemsA[b]).wait()
        pltpu.make_async_copy(bufsB[b], o_hbm.at[0, 0, :, pl.ds(0, LH)], osemsB[b]).wait()

    load(0, 0)
    for m in range(NITEMS):
        b = m % 2
        c, _hp, g = item_coords(m)
        gbase = g * LH
        wait_load(b)

        @pl.loop(0, LH // L)
        def _(kk):
            sl = pl.ds(kk * L, L)
            gsl = pl.ds(gbase + kk * L, L)
            tv = tf_v[gsl]
            m1 = tv == 1
            m2 = tv == 2
            m3 = tv == 3
            mn = cons_v[c, 0, gsl]
            sc = cons_v[c, 1, gsl]
            bA = bufsA[b]
            bB = bufsB[b]

            @pl.loop(0, HALF, step=4)
            def _(w0):
                for i in range(4):
                    w = w0 + i
                    mw = W - 1 - w
                    aw = bA[w, sl]
                    am = bA[mw, sl]
                    bw = bB[w, sl]
                    bm = bB[mw, sl]
                    bA[w, sl] = jnp.where(
                        m3, (aw - mn) * sc,
                        jnp.where(m2, bw, jnp.where(m1, am, aw)))
                    bA[mw, sl] = jnp.where(
                        m3, (am - mn) * sc,
                        jnp.where(m2, bm, jnp.where(m1, aw, am)))
                    bB[w, sl] = jnp.where(
                        m3, (bw - mn) * sc,
                        jnp.where(m2, aw, jnp.where(m1, bm, bw)))
                    bB[mw, sl] = jnp.where(
                        m3, (bm - mn) * sc,
                        jnp.where(m2, am, jnp.where(m1, bw, bm)))

        if m >= 1:
            wait_store(1 - b)
        if m + 1 < NITEMS:
            load(m + 1, 1 - b)
        store(m, b)

    wait_store((NITEMS - 1) % 2)


@jax.jit
def kernel(x, sampled_tf):
    xt = jnp.transpose(x, (1, 2, 3, 0))
    mesh = plsc.VectorSubcoreMesh(
        core_axis_name="c", subcore_axis_name="s", num_cores=NC, num_subcores=NS
    )
    cp = pltpu.CompilerParams()
    if "needs_layout_passes" in pltpu.CompilerParams.__dataclass_fields__:
        cp = dataclasses.replace(cp, needs_layout_passes=False)
    part_sc = pl.kernel(
        _minmax_body,
        out_type=jax.ShapeDtypeStruct((NC, 2, B), jnp.float32),
        mesh=mesh,
        scratch_types=[
            pltpu.VMEM((W, B), jnp.float32),
            pltpu.VMEM((W, B), jnp.float32),
            pltpu.VMEM((2, B), jnp.float32),
            pltpu.VMEM((2, B), jnp.float32),
            pltpu.VMEM_SHARED((NS, 2, B), jnp.float32),
            pltpu.SemaphoreType.DMA,
            pltpu.SemaphoreType.DMA,
        ],
        compiler_params=cp,
    )(xt)
    part_tc = pl.pallas_call(
        _tc_minmax_body,
        grid=(C - 1, H // 8),
        in_specs=[
            pl.BlockSpec((1, 8, W, B), lambda c, h: (c + 1, h, 0, 0)),
        ],
        out_specs=pl.BlockSpec((1, 2, B), lambda c, h: (c, 0, 0)),
        out_shape=jax.ShapeDtypeStruct((C - 1, 2, B), jnp.float32),
    )(xt)
    ot = pl.kernel(
        _apply_body,
        out_type=jax.ShapeDtypeStruct((C, H, W, B), jnp.float32),
        mesh=mesh,
        scratch_types=[
            pltpu.VMEM((W, LH), jnp.float32),
            pltpu.VMEM((W, LH), jnp.float32),
            pltpu.VMEM((W, LH), jnp.float32),
            pltpu.VMEM((W, LH), jnp.float32),
            pltpu.VMEM((C, 2, B), jnp.float32),
            pltpu.VMEM((NC, 2, B), jnp.float32),
            pltpu.VMEM((C - 1, 2, B), jnp.float32),
            pltpu.VMEM((B,), jnp.int32),
            pltpu.SemaphoreType.DMA,
            pltpu.SemaphoreType.DMA,
            pltpu.SemaphoreType.DMA,
            pltpu.SemaphoreType.DMA,
            pltpu.SemaphoreType.DMA,
            pltpu.SemaphoreType.DMA,
            pltpu.SemaphoreType.DMA,
            pltpu.SemaphoreType.DMA,
        ],
        compiler_params=cp,
    )(xt, sampled_tf, part_sc, part_tc)
    return jnp.transpose(ot, (3, 0, 1, 2))


# speedup vs baseline: 1.1364x; 1.0005x over previous
import dataclasses

import jax
import jax.numpy as jnp
from jax import lax
from jax.experimental import pallas as pl
from jax.experimental.pallas import tpu as pltpu
from jax.experimental.pallas import tpu_sc as plsc

NB_TF = 4
B, C, H, W = 256, 3, 224, 224
L = 16
NC, NS = 2, 16
NW = NC * NS
KCH = B // L
HPT = H // NW
HALF = H // 2


SC_MM_SLABS = 320
SC_MM_PER_TILE = SC_MM_SLABS // NW
TC_C1_BLOCKS = (H - 96) // 8


def _minmax_body(xt_hbm, part_hbm, buf0, buf1, acc_v, stage_v, shared_v,
                 isem0, isem1):
    cid = lax.axis_index("c")
    sid = lax.axis_index("s")
    wid = cid * NS + sid

    bufs = (buf0, buf1)
    isems = (isem0, isem1)

    def load(j, b):
        s = wid * SC_MM_PER_TILE + j

        @pl.when(s < H)
        def _():
            pltpu.async_copy(xt_hbm.at[0, s], bufs[b], isems[b])

        @pl.when(s >= H)
        def _():
            pltpu.async_copy(xt_hbm.at[1, s - H], bufs[b], isems[b])

    def wait_load(b):
        pltpu.make_async_copy(xt_hbm.at[0, 0], bufs[b], isems[b]).wait()

    for ch in range(2):
        @pl.loop(0, KCH)
        def _(k):
            acc_v[ch, 0, pl.ds(k * L, L)] = jnp.full((L,), jnp.inf, jnp.float32)
            acc_v[ch, 1, pl.ds(k * L, L)] = jnp.full((L,), -jnp.inf, jnp.float32)

    load(0, 0)
    for j in range(SC_MM_PER_TILE):
        b = j % 2
        wait_load(b)
        if j + 1 < SC_MM_PER_TILE:
            load(j + 1, 1 - b)
        ch = jnp.where(wid * SC_MM_PER_TILE + j >= H, 1, 0)

        @pl.loop(0, KCH)
        def _(k):
            sl = pl.ds(k * L, L)

            @pl.loop(0, H, step=16)
            def _(w0):
                vs = [bufs[b][w0 + i, sl] for i in range(16)]
                mn = vs[0]
                mx = vs[0]
                for v in vs[1:]:
                    mn = jnp.minimum(mn, v)
                    mx = jnp.maximum(mx, v)
                acc_v[ch, 0, sl] = jnp.minimum(acc_v[ch, 0, sl], mn)
                acc_v[ch, 1, sl] = jnp.maximum(acc_v[ch, 1, sl], mx)

    pltpu.sync_copy(acc_v, shared_v.at[sid])
    plsc.subcore_barrier()

    @pl.when(sid == 0)
    def _():
        @pl.loop(0, NS)
        def _(i):
            pltpu.sync_copy(shared_v.at[i], stage_v)
            for ch in range(2):
                @pl.loop(0, KCH)
                def _(k):
                    sl = pl.ds(k * L, L)
                    acc_v[ch, 0, sl] = jnp.minimum(acc_v[ch, 0, sl], stage_v[ch, 0, sl])
                    acc_v[ch, 1, sl] = jnp.maximum(acc_v[ch, 1, sl], stage_v[ch, 1, sl])

        pltpu.sync_copy(acc_v, part_hbm.at[cid])


def _tc_minmax_body(x_ref, o_ref):
    i = pl.program_id(0)
    v = x_ref[0]
    mn = jnp.min(v, axis=(0, 1))
    mx = jnp.max(v, axis=(0, 1))
    first = (i == 0) | (i == TC_C1_BLOCKS)

    @pl.when(first)
    def _():
        o_ref[0, 0, :] = mn
        o_ref[0, 1, :] = mx

    @pl.when(~first)
    def _():
        o_ref[0, 0, :] = jnp.minimum(o_ref[0, 0, :], mn)
        o_ref[0, 1, :] = jnp.maximum(o_ref[0, 1, :], mx)


LH = B // 2
ITEMS_PER_C = 2 * HALF // NW


def _apply_body(xt_hbm, tf_hbm, psc_hbm, ptc_hbm, o_hbm,
                bufA0, bufB0, bufA1, bufB1, cons_v, stage_sc, stage_tc, tf_v,
                iA0, iB0, iA1, iB1, oA0, oB0, oA1, oB1):
    cid = lax.axis_index("c")
    sid = lax.axis_index("s")
    wid = cid * NS + sid

    bufsA = (bufA0, bufA1)
    bufsB = (bufB0, bufB1)
    isemsA = (iA0, iA1)
    isemsB = (iB0, iB1)
    osemsA = (oA0, oA1)
    osemsB = (oB0, oB1)

    pltpu.sync_copy(tf_hbm, tf_v)
    pltpu.sync_copy(psc_hbm, stage_sc)
    pltpu.sync_copy(ptc_hbm, stage_tc)

    @pl.loop(0, KCH)
    def _(k):
        sl = pl.ds(k * L, L)
        ones = jnp.full((L,), 1.0, jnp.float32)
        eps = jnp.full((L,), 1e-6, jnp.float32)
        mn = jnp.minimum(stage_sc[0, 0, 0, sl], stage_sc[1, 0, 0, sl])
        mx = jnp.maximum(stage_sc[0, 0, 1, sl], stage_sc[1, 0, 1, sl])
        cons_v[0, 0, sl] = mn
        cons_v[0, 1, sl] = ones / jnp.maximum(mx - mn, eps)
        mn = jnp.minimum(jnp.minimum(stage_sc[0, 1, 0, sl], stage_sc[1, 1, 0, sl]),
                         stage_tc[0, 0, sl])
        mx = jnp.maximum(jnp.maximum(stage_sc[0, 1, 1, sl], stage_sc[1, 1, 1, sl]),
                         stage_tc[0, 1, sl])
        cons_v[1, 0, sl] = mn
        cons_v[1, 1, sl] = ones / jnp.maximum(mx - mn, eps)
        mn = stage_tc[1, 0, sl]
        mx = stage_tc[1, 1, sl]
        cons_v[2, 0, sl] = mn
        cons_v[2, 1, sl] = ones / jnp.maximum(mx - mn, eps)

    NITEMS = C * ITEMS_PER_C

    def item_coords(m):
        c = m // ITEMS_PER_C
        q = wid + NW * (m % ITEMS_PER_C)
        hp = lax.shift_right_logical(q, 1)
        g = lax.bitwise_and(q, 1)
        return c, hp, g

    def load(m, b):
        c, hp, g = item_coords(m)
        col = pl.ds(g * LH, LH)
        pltpu.async_copy(xt_hbm.at[c, hp, :, col], bufsA[b], isemsA[b])
        pltpu.async_copy(xt_hbm.at[c, H - 1 - hp, :, col], bufsB[b], isemsB[b])

    def store(m, b):
        c, hp, g = item_coords(m)
        col = pl.ds(g * LH, LH)
        pltpu.async_copy(bufsA[b], o_hbm.at[c, hp, :, col], osemsA[b])
        pltpu.async_copy(bufsB[b], o_hbm.at[c, H - 1 - hp, :, col], osemsB[b])

    def wait_load(b):
        pltpu.make_async_copy(xt_hbm.at[0, 0, :, pl.ds(0, LH)], bufsA[b], isemsA[b]).wait()
        pltpu.make_async_copy(xt_hbm.at[0, 0, :, pl.ds(0, LH)], bufsB[b], isemsB[b]).wait()

    def wait_store(b):
        pltpu.make_async_copy(bufsA[b], o_hbm.at[0, 0, :, pl.ds(0, LH)], osemsA[b]).wait()
        pltpu.make_async_copy(bufsB[b], o_hbm.at[0, 0, :, pl.ds(0, LH)], osemsB[b]).wait()

    load(0, 0)
    for m in range(NITEMS):
        b = m % 2
        c, _hp, g = item_coords(m)
        gbase = g * LH
        wait_load(b)

        @pl.loop(0, LH // L)
        def _(kk):
            sl = pl.ds(kk * L, L)
            gsl = pl.ds(gbase + kk * L, L)
            tv = tf_v[gsl]
            m1 = tv == 1
            m2 = tv == 2
            m3 = tv == 3
            mn = cons_v[c, 0, gsl]
            sc = cons_v[c, 1, gsl]
            bA = bufsA[b]
            bB = bufsB[b]

            @pl.loop(0, HALF, step=4)
            def _(w0):
                for i in range(4):
                    w = w0 + i
                    mw = W - 1 - w
                    aw = bA[w, sl]
                    am = bA[mw, sl]
                    bw = bB[w, sl]
                    bm = bB[mw, sl]
                    bA[w, sl] = jnp.where(
                        m3, (aw - mn) * sc,
                        jnp.where(m2, bw, jnp.where(m1, am, aw)))
                    bA[mw, sl] = jnp.where(
                        m3, (am - mn) * sc,
                        jnp.where(m2, bm, jnp.where(m1, aw, am)))
                    bB[w, sl] = jnp.where(
                        m3, (bw - mn) * sc,
                        jnp.where(m2, aw, jnp.where(m1, bm, bw)))
                    bB[mw, sl] = jnp.where(
                        m3, (bm - mn) * sc,
                        jnp.where(m2, am, jnp.where(m1, bw, bm)))

        if m >= 1:
            wait_store(1 - b)
        if m + 1 < NITEMS:
            load(m + 1, 1 - b)
        store(m, b)

    wait_store((NITEMS - 1) % 2)


@jax.jit
def kernel(x, sampled_tf):
    xt = jnp.transpose(x, (1, 2, 3, 0))
    mesh = plsc.VectorSubcoreMesh(
        core_axis_name="c", subcore_axis_name="s", num_cores=NC, num_subcores=NS
    )
    cp = pltpu.CompilerParams()
    if "needs_layout_passes" in pltpu.CompilerParams.__dataclass_fields__:
        cp = dataclasses.replace(cp, needs_layout_passes=False)
    part_sc = pl.kernel(
        _minmax_body,
        out_type=jax.ShapeDtypeStruct((NC, 2, 2, B), jnp.float32),
        mesh=mesh,
        scratch_types=[
            pltpu.VMEM((W, B), jnp.float32),
            pltpu.VMEM((W, B), jnp.float32),
            pltpu.VMEM((2, 2, B), jnp.float32),
            pltpu.VMEM((2, 2, B), jnp.float32),
            pltpu.VMEM_SHARED((NS, 2, 2, B), jnp.float32),
            pltpu.SemaphoreType.DMA,
            pltpu.SemaphoreType.DMA,
        ],
        compiler_params=cp,
    )(xt)
    part_tc = pl.pallas_call(
        _tc_minmax_body,
        grid=(TC_C1_BLOCKS + H // 8,),
        in_specs=[
            pl.BlockSpec(
                (1, 8, W, B),
                lambda i: (jnp.where(i < TC_C1_BLOCKS, 1, 2),
                           jnp.where(i < TC_C1_BLOCKS, i + 96 // 8,
                                     i - TC_C1_BLOCKS), 0, 0),
            ),
        ],
        out_specs=pl.BlockSpec(
            (1, 2, B), lambda i: (jnp.where(i < TC_C1_BLOCKS, 0, 1), 0, 0)),
        out_shape=jax.ShapeDtypeStruct((C - 1, 2, B), jnp.float32),
    )(xt)
    ot = pl.kernel(
        _apply_body,
        out_type=jax.ShapeDtypeStruct((C, H, W, B), jnp.float32),
        mesh=mesh,
        scratch_types=[
            pltpu.VMEM((W, LH), jnp.float32),
            pltpu.VMEM((W, LH), jnp.float32),
            pltpu.VMEM((W, LH), jnp.float32),
            pltpu.VMEM((W, LH), jnp.float32),
            pltpu.VMEM((C, 2, B), jnp.float32),
            pltpu.VMEM((NC, 2, 2, B), jnp.float32),
            pltpu.VMEM((C - 1, 2, B), jnp.float32),
            pltpu.VMEM((B,), jnp.int32),
            pltpu.SemaphoreType.DMA,
            pltpu.SemaphoreType.DMA,
            pltpu.SemaphoreType.DMA,
            pltpu.SemaphoreType.DMA,
            pltpu.SemaphoreType.DMA,
            pltpu.SemaphoreType.DMA,
            pltpu.SemaphoreType.DMA,
            pltpu.SemaphoreType.DMA,
        ],
        compiler_params=cp,
    )(xt, sampled_tf, part_sc, part_tc)
    return jnp.transpose(ot, (3, 0, 1, 2))
